# baseline (speedup 1.0000x reference)
import functools

import jax
import jax.numpy as jnp
from jax import lax
from jax.experimental import pallas as pl
from jax.experimental.pallas import tpu as pltpu
from jax.experimental.pallas import tpu_sc as plsc

N_NODES = 10000
N_EDGES = 320000
D = 128
NREL = 1000

NC = 2
NS = 16
NW = NC * NS
CHUNK = 80
EPT = N_EDGES // NW
CPT = EPT // CHUNK

_mesh = plsc.VectorSubcoreMesh(core_axis_name="c", subcore_axis_name="s")


@functools.partial(
    pl.kernel,
    mesh=_mesh,
    out_type=(
        jax.ShapeDtypeStruct((N_EDGES,), jnp.float32),
        jax.ShapeDtypeStruct((N_EDGES, D), jnp.float32),
        jax.ShapeDtypeStruct((N_EDGES, D), jnp.float32),
        jax.ShapeDtypeStruct((N_EDGES, D), jnp.float32),
    ),
    scratch_types=(
        [pltpu.VMEM((CPT, CHUNK), jnp.int32)] * 3
        + [pltpu.VMEM((CHUNK, D), jnp.float32)] * 6
        + [pltpu.VMEM((CHUNK,), jnp.float32)] * 2
        + [pltpu.SemaphoreType.DMA] * 4
    ),
)
def _distmult_sc(z_hbm, src_hbm, dst_hbm, typ_hbm, rel_hbm,
                 score_hbm, zsrc_hbm, relo_hbm, zdst_hbm,
                 src_idx, dst_idx, typ_idx,
                 s0, d0, r0, s1, d1, r1,
                 sc0, sc1,
                 gsem0, gsem1, wsem0, wsem1):
    sid = lax.axis_index("s")
    wid = sid * NC + lax.axis_index("c")

    bufs = ((s0, d0, r0), (s1, d1, r1))
    scs = (sc0, sc1)
    gsems = (gsem0, gsem1)
    wsems = (wsem0, wsem1)

    pltpu.sync_copy(src_hbm.at[wid], src_idx)
    pltpu.sync_copy(dst_hbm.at[wid], dst_idx)
    pltpu.sync_copy(typ_hbm.at[wid], typ_idx)

    lanes = lax.iota(jnp.int32, 16)

    def start_gathers(c, b):
        s_r, d_r, r_r = bufs[b]
        pltpu.async_copy(z_hbm.at[src_idx.at[c]], s_r, gsems[b])
        pltpu.async_copy(z_hbm.at[dst_idx.at[c]], d_r, gsems[b])
        pltpu.async_copy(rel_hbm.at[typ_idx.at[c]], r_r, gsems[b])

    def wait_gathers(b):
        s_r, d_r, r_r = bufs[b]
        pltpu.make_async_copy(z_hbm.at[pl.ds(0, CHUNK)], s_r, gsems[b]).wait()
        pltpu.make_async_copy(z_hbm.at[pl.ds(0, CHUNK)], d_r, gsems[b]).wait()
        pltpu.make_async_copy(rel_hbm.at[pl.ds(0, CHUNK)], r_r, gsems[b]).wait()

    def start_writes(c, b):
        s_r, d_r, r_r = bufs[b]
        base = wid * EPT + c * CHUNK
        pltpu.async_copy(s_r, zsrc_hbm.at[pl.ds(base, CHUNK)], wsems[b])
        pltpu.async_copy(r_r, relo_hbm.at[pl.ds(base, CHUNK)], wsems[b])
        pltpu.async_copy(d_r, zdst_hbm.at[pl.ds(base, CHUNK)], wsems[b])
        pltpu.async_copy(scs[b], score_hbm.at[pl.ds(base, CHUNK)], wsems[b])

    def wait_writes(b):
        s_r, d_r, r_r = bufs[b]
        pltpu.make_async_copy(s_r, zsrc_hbm.at[pl.ds(0, CHUNK)], wsems[b]).wait()
        pltpu.make_async_copy(r_r, relo_hbm.at[pl.ds(0, CHUNK)], wsems[b]).wait()
        pltpu.make_async_copy(d_r, zdst_hbm.at[pl.ds(0, CHUNK)], wsems[b]).wait()
        pltpu.make_async_copy(scs[b], score_hbm.at[pl.ds(0, CHUNK)], wsems[b]).wait()

    def compute_range(b, eb_lo, eb_hi):
        s_r, d_r, r_r = bufs[b]
        for eb in range(eb_lo, eb_hi):
            def edge_body(i, svec, eb=eb):
                e = eb * 16 + i
                acc = jnp.zeros((16,), jnp.float32)
                for j in range(D // 16):
                    sv = s_r[e, pl.ds(j * 16, 16)]
                    rv = r_r[e, pl.ds(j * 16, 16)]
                    dv = d_r[e, pl.ds(j * 16, 16)]
                    acc = acc + sv * rv * dv
                for st in (1, 2, 4, 8):
                    acc = acc + acc.at[lanes ^ st].get(mode="promise_in_bounds")
                return jnp.where(lanes == i, acc, svec)

            svec = lax.fori_loop(0, 16, edge_body, jnp.zeros((16,), jnp.float32))
            scs[b][pl.ds(eb * 16, 16)] = svec

    NEB = CHUNK // 16

    def process(c, b, first=False, last=False):
        wait_gathers(b)
        compute_range(b, 0, NEB // 2)
        if not first:
            wait_writes(1 - b)
        if not last:
            start_gathers(c + 1, 1 - b)
        compute_range(b, NEB // 2, NEB)
        start_writes(c, b)

    start_gathers(0, 0)
    process(0, 0, first=True)

    def pair_body(k, carry):
        process(2 * k + 1, 1)
        process(2 * k + 2, 0)
        return carry

    lax.fori_loop(0, (CPT - 3) // 2, pair_body, 0)
    process(CPT - 2, 1)
    process(CPT - 1, 0, last=True)
    wait_writes(0)


def kernel(z, edge_index, edge_type, rel_emb):
    src = edge_index[0].astype(jnp.int32).reshape(NW, CPT, CHUNK)
    dst = edge_index[1].astype(jnp.int32).reshape(NW, CPT, CHUNK)
    typ = edge_type.astype(jnp.int32).reshape(NW, CPT, CHUNK)
    score, z_src, rel, z_dst = _distmult_sc(z, src, dst, typ, rel_emb)
    return score, z_src, rel, z_dst


# speedup vs baseline: 1.0623x; 1.0623x over previous
import functools

import jax
import jax.numpy as jnp
from jax import lax
from jax.experimental import pallas as pl
from jax.experimental.pallas import tpu as pltpu
from jax.experimental.pallas import tpu_sc as plsc

N_NODES = 10000
N_EDGES = 320000
D = 128
NREL = 1000

NC = 2
NS = 16
NW = NC * NS
CHUNK = 80
EPT = N_EDGES // NW
CPT = EPT // CHUNK
NB = 3

_mesh = plsc.VectorSubcoreMesh(core_axis_name="c", subcore_axis_name="s")


@functools.partial(
    pl.kernel,
    mesh=_mesh,
    out_type=(
        jax.ShapeDtypeStruct((N_EDGES,), jnp.float32),
        jax.ShapeDtypeStruct((N_EDGES, D), jnp.float32),
        jax.ShapeDtypeStruct((N_EDGES, D), jnp.float32),
        jax.ShapeDtypeStruct((N_EDGES, D), jnp.float32),
    ),
    scratch_types=(
        [pltpu.VMEM((CHUNK, D), jnp.float32)] * 9
        + [pltpu.VMEM((CHUNK,), jnp.int32)] * 9
        + [pltpu.VMEM((CHUNK,), jnp.float32)] * 3
        + [pltpu.SemaphoreType.DMA] * 9
    ),
)
def _distmult_sc(z_hbm, src_hbm, dst_hbm, typ_hbm, rel_hbm,
                 score_hbm, zsrc_hbm, relo_hbm, zdst_hbm,
                 s0, d0, r0, s1, d1, r1, s2, d2, r2,
                 is0, id0, it0, is1, id1, it1, is2, id2, it2,
                 sc0, sc1, sc2,
                 gsem0, gsem1, gsem2, wsem0, wsem1, wsem2,
                 isem0, isem1, isem2):
    sid = lax.axis_index("s")
    wid = sid * NC + lax.axis_index("c")

    bufs = ((s0, d0, r0), (s1, d1, r1), (s2, d2, r2))
    idxs = ((is0, id0, it0), (is1, id1, it1), (is2, id2, it2))
    scs = (sc0, sc1, sc2)
    gsems = (gsem0, gsem1, gsem2)
    wsems = (wsem0, wsem1, wsem2)
    isems = (isem0, isem1, isem2)

    lanes = lax.iota(jnp.int32, 16)

    def start_idx(c, b):
        i_s, i_d, i_t = idxs[b]
        pltpu.async_copy(src_hbm.at[wid, c], i_s, isems[b])
        pltpu.async_copy(dst_hbm.at[wid, c], i_d, isems[b])
        pltpu.async_copy(typ_hbm.at[wid, c], i_t, isems[b])

    def wait_idx(b):
        i_s, i_d, i_t = idxs[b]
        pltpu.make_async_copy(src_hbm.at[0, 0], i_s, isems[b]).wait()
        pltpu.make_async_copy(dst_hbm.at[0, 0], i_d, isems[b]).wait()
        pltpu.make_async_copy(typ_hbm.at[0, 0], i_t, isems[b]).wait()

    def start_gathers(b):
        s_r, d_r, r_r = bufs[b]
        i_s, i_d, i_t = idxs[b]
        pltpu.async_copy(z_hbm.at[i_s], s_r, gsems[b])
        pltpu.async_copy(z_hbm.at[i_d], d_r, gsems[b])
        pltpu.async_copy(rel_hbm.at[i_t], r_r, gsems[b])

    def wait_gathers(b):
        s_r, d_r, r_r = bufs[b]
        pltpu.make_async_copy(z_hbm.at[pl.ds(0, CHUNK)], s_r, gsems[b]).wait()
        pltpu.make_async_copy(z_hbm.at[pl.ds(0, CHUNK)], d_r, gsems[b]).wait()
        pltpu.make_async_copy(rel_hbm.at[pl.ds(0, CHUNK)], r_r, gsems[b]).wait()

    def start_writes(c, b):
        s_r, d_r, r_r = bufs[b]
        base = wid * EPT + c * CHUNK
        pltpu.async_copy(s_r, zsrc_hbm.at[pl.ds(base, CHUNK)], wsems[b])
        pltpu.async_copy(r_r, relo_hbm.at[pl.ds(base, CHUNK)], wsems[b])
        pltpu.async_copy(d_r, zdst_hbm.at[pl.ds(base, CHUNK)], wsems[b])
        pltpu.async_copy(scs[b], score_hbm.at[pl.ds(base, CHUNK)], wsems[b])

    def wait_writes(b):
        s_r, d_r, r_r = bufs[b]
        pltpu.make_async_copy(s_r, zsrc_hbm.at[pl.ds(0, CHUNK)], wsems[b]).wait()
        pltpu.make_async_copy(r_r, relo_hbm.at[pl.ds(0, CHUNK)], wsems[b]).wait()
        pltpu.make_async_copy(d_r, zdst_hbm.at[pl.ds(0, CHUNK)], wsems[b]).wait()
        pltpu.make_async_copy(scs[b], score_hbm.at[pl.ds(0, CHUNK)], wsems[b]).wait()

    def compute(b):
        s_r, d_r, r_r = bufs[b]
        for eb in range(CHUNK // 16):
            def edge_body(i, svec, eb=eb):
                e = eb * 16 + i
                acc = jnp.zeros((16,), jnp.float32)
                for j in range(D // 16):
                    sv = s_r[e, pl.ds(j * 16, 16)]
                    rv = r_r[e, pl.ds(j * 16, 16)]
                    dv = d_r[e, pl.ds(j * 16, 16)]
                    acc = acc + sv * rv * dv
                for st in (1, 2, 4, 8):
                    acc = acc + acc.at[lanes ^ st].get(mode="promise_in_bounds")
                return jnp.where(lanes == i, acc, svec)

            svec = lax.fori_loop(0, 16, edge_body, jnp.zeros((16,), jnp.float32))
            scs[b][pl.ds(eb * 16, 16)] = svec

    def process(c, b, first=False, last=False, prefetch=True):
        bn = (b + 1) % NB
        if not first:
            wait_writes(bn)
        if not last:
            wait_idx(bn)
            start_gathers(bn)
        wait_gathers(b)
        if prefetch:
            start_idx(c + 3, b)
        compute(b)
        start_writes(c, b)

    start_idx(0, 0)
    start_idx(1, 1)
    start_idx(2, 2)
    wait_idx(0)
    start_gathers(0)
    process(0, 0, first=True)
    process(1, 1, first=True)

    def tri_body(k, carry):
        process(3 * k + 2, 2)
        process(3 * k + 3, 0)
        process(3 * k + 4, 1)
        return carry

    lax.fori_loop(0, (CPT - 5) // 3, tri_body, 0)
    process(CPT - 3, 2, prefetch=False)
    process(CPT - 2, 0, prefetch=False)
    process(CPT - 1, 1, last=True, prefetch=False)
    wait_writes(0)
    wait_writes(1)


def kernel(z, edge_index, edge_type, rel_emb):
    src = edge_index[0].astype(jnp.int32).reshape(NW, CPT, CHUNK)
    dst = edge_index[1].astype(jnp.int32).reshape(NW, CPT, CHUNK)
    typ = edge_type.astype(jnp.int32).reshape(NW, CPT, CHUNK)
    score, z_src, rel, z_dst = _distmult_sc(z, src, dst, typ, rel_emb)
    return score, z_src, rel, z_dst


# speedup vs baseline: 1.0651x; 1.0026x over previous
import functools

import jax
import jax.numpy as jnp
from jax import lax
from jax.experimental import pallas as pl
from jax.experimental.pallas import tpu as pltpu
from jax.experimental.pallas import tpu_sc as plsc

N_NODES = 10000
N_EDGES = 320000
D = 128
NREL = 1000

NC = 2
NS = 16
NW = NC * NS
CHUNK = 80
EPT = N_EDGES // NW
CPT = EPT // CHUNK
NB = 3

_mesh = plsc.VectorSubcoreMesh(core_axis_name="c", subcore_axis_name="s")


@functools.partial(
    pl.kernel,
    mesh=_mesh,
    out_type=(
        jax.ShapeDtypeStruct((N_EDGES,), jnp.float32),
        jax.ShapeDtypeStruct((N_EDGES, D), jnp.float32),
        jax.ShapeDtypeStruct((N_EDGES, D), jnp.float32),
        jax.ShapeDtypeStruct((N_EDGES, D), jnp.float32),
    ),
    scratch_types=(
        [pltpu.VMEM((CHUNK, D), jnp.float32)] * 9
        + [pltpu.VMEM((CHUNK,), jnp.int32)] * 9
        + [pltpu.VMEM((EPT,), jnp.float32)]
        + [pltpu.SemaphoreType.DMA] * 10
    ),
)
def _distmult_sc(z_hbm, src_hbm, dst_hbm, typ_hbm, rel_hbm,
                 score_hbm, zsrc_hbm, relo_hbm, zdst_hbm,
                 s0, d0, r0, s1, d1, r1, s2, d2, r2,
                 is0, id0, it0, is1, id1, it1, is2, id2, it2,
                 sc_all,
                 gsem0, gsem1, gsem2, wsem0, wsem1, wsem2,
                 isem0, isem1, isem2, ssem):
    sid = lax.axis_index("s")
    wid = sid * NC + lax.axis_index("c")

    bufs = ((s0, d0, r0), (s1, d1, r1), (s2, d2, r2))
    idxs = ((is0, id0, it0), (is1, id1, it1), (is2, id2, it2))
    gsems = (gsem0, gsem1, gsem2)
    wsems = (wsem0, wsem1, wsem2)
    isems = (isem0, isem1, isem2)

    lanes = lax.iota(jnp.int32, 16)

    def start_idx(c, b):
        i_s, i_d, i_t = idxs[b]
        pltpu.async_copy(src_hbm.at[wid, c], i_s, isems[b])
        pltpu.async_copy(dst_hbm.at[wid, c], i_d, isems[b])
        pltpu.async_copy(typ_hbm.at[wid, c], i_t, isems[b])

    def wait_idx(b):
        i_s, i_d, i_t = idxs[b]
        pltpu.make_async_copy(src_hbm.at[0, 0], i_s, isems[b]).wait()
        pltpu.make_async_copy(dst_hbm.at[0, 0], i_d, isems[b]).wait()
        pltpu.make_async_copy(typ_hbm.at[0, 0], i_t, isems[b]).wait()

    def start_gathers(b):
        s_r, d_r, r_r = bufs[b]
        i_s, i_d, i_t = idxs[b]
        pltpu.async_copy(z_hbm.at[i_s], s_r, gsems[b])
        pltpu.async_copy(z_hbm.at[i_d], d_r, gsems[b])
        pltpu.async_copy(rel_hbm.at[i_t], r_r, gsems[b])

    def wait_gathers(b):
        s_r, d_r, r_r = bufs[b]
        pltpu.make_async_copy(z_hbm.at[pl.ds(0, CHUNK)], s_r, gsems[b]).wait()
        pltpu.make_async_copy(z_hbm.at[pl.ds(0, CHUNK)], d_r, gsems[b]).wait()
        pltpu.make_async_copy(rel_hbm.at[pl.ds(0, CHUNK)], r_r, gsems[b]).wait()

    def start_writes(c, b):
        s_r, d_r, r_r = bufs[b]
        base = wid * EPT + c * CHUNK
        pltpu.async_copy(s_r, zsrc_hbm.at[pl.ds(base, CHUNK)], wsems[b])
        pltpu.async_copy(r_r, relo_hbm.at[pl.ds(base, CHUNK)], wsems[b])
        pltpu.async_copy(d_r, zdst_hbm.at[pl.ds(base, CHUNK)], wsems[b])

    def wait_writes(b):
        s_r, d_r, r_r = bufs[b]
        pltpu.make_async_copy(s_r, zsrc_hbm.at[pl.ds(0, CHUNK)], wsems[b]).wait()
        pltpu.make_async_copy(r_r, relo_hbm.at[pl.ds(0, CHUNK)], wsems[b]).wait()
        pltpu.make_async_copy(d_r, zdst_hbm.at[pl.ds(0, CHUNK)], wsems[b]).wait()

    def compute(c, b):
        s_r, d_r, r_r = bufs[b]
        for eb in range(CHUNK // 16):
            def edge_body(i, svec, eb=eb):
                e = eb * 16 + i
                acc = jnp.zeros((16,), jnp.float32)
                for j in range(D // 16):
                    sv = s_r[e, pl.ds(j * 16, 16)]
                    rv = r_r[e, pl.ds(j * 16, 16)]
                    dv = d_r[e, pl.ds(j * 16, 16)]
                    acc = acc + sv * rv * dv
                for st in (1, 2, 4, 8):
                    acc = acc + acc.at[lanes ^ st].get(mode="promise_in_bounds")
                return jnp.where(lanes == i, acc, svec)

            svec = lax.fori_loop(0, 16, edge_body, jnp.zeros((16,), jnp.float32))
            sc_all[pl.ds(c * CHUNK + eb * 16, 16)] = svec

    def process(c, b, first=False, last=False, prefetch=True):
        bn = (b + 1) % NB
        if not first:
            wait_writes(bn)
        if not last:
            wait_idx(bn)
            start_gathers(bn)
        wait_gathers(b)
        start_writes(c, b)
        if prefetch:
            start_idx(c + 3, b)
        compute(c, b)

    start_idx(0, 0)
    start_idx(1, 1)
    start_idx(2, 2)
    wait_idx(0)
    start_gathers(0)
    process(0, 0, first=True)
    process(1, 1, first=True)

    def tri_body(k, carry):
        process(3 * k + 2, 2)
        process(3 * k + 3, 0)
        process(3 * k + 4, 1)
        return carry

    lax.fori_loop(0, (CPT - 5) // 3, tri_body, 0)
    process(CPT - 3, 2, prefetch=False)
    process(CPT - 2, 0, prefetch=False)
    process(CPT - 1, 1, last=True, prefetch=False)
    pltpu.async_copy(sc_all, score_hbm.at[pl.ds(wid * EPT, EPT)], ssem)
    wait_writes(0)
    wait_writes(1)
    pltpu.make_async_copy(sc_all, score_hbm.at[pl.ds(0, EPT)], ssem).wait()


def kernel(z, edge_index, edge_type, rel_emb):
    src = edge_index[0].astype(jnp.int32).reshape(NW, CPT, CHUNK)
    dst = edge_index[1].astype(jnp.int32).reshape(NW, CPT, CHUNK)
    typ = edge_type.astype(jnp.int32).reshape(NW, CPT, CHUNK)
    score, z_src, rel, z_dst = _distmult_sc(z, src, dst, typ, rel_emb)
    return score, z_src, rel, z_dst
